# baseline (speedup 1.0000x reference)
import functools

import jax
import jax.numpy as jnp
from jax import lax
from jax.experimental import pallas as pl
from jax.experimental.pallas import tpu as pltpu
from jax.experimental.pallas import tpu_sc as plsc

_L = 16
_NC = 2
_NS = 16
_NW = _NC * _NS


def _sc_add(x2, pos2, *, rpw, wpb, rc, nb, d):
    mesh = plsc.VectorSubcoreMesh(core_axis_name="c", subcore_axis_name="s")
    vpr = d // _L

    @functools.partial(
        pl.kernel,
        mesh=mesh,
        out_type=jax.ShapeDtypeStruct(x2.shape, jnp.float32),
        scratch_types=[
            pltpu.VMEM((rc, d), jnp.float32),
            pltpu.VMEM((rc, d), jnp.float32),
        ],
    )
    def k(x_hbm, pos_hbm, out_hbm, xbuf, pbuf):
        c = lax.axis_index("c")
        s = lax.axis_index("s")
        w = s * _NC + c
        xr0 = w * rpw
        pr0 = (w % wpb) * rpw

        def block(j, carry):
            bx = xr0 + j * rc
            bp = pr0 + j * rc
            pltpu.sync_copy(x_hbm.at[pl.ds(bx, rc)], xbuf)
            pltpu.sync_copy(pos_hbm.at[pl.ds(bp, rc)], pbuf)

            def row_add(r, carry2):
                for kk in range(vpr):
                    sl = pl.ds(kk * _L, _L)
                    xbuf[r, sl] = xbuf[r, sl] + pbuf[r, sl]
                return carry2

            lax.fori_loop(0, rc, row_add, 0)
            pltpu.sync_copy(xbuf, out_hbm.at[pl.ds(bx, rc)])
            return carry

        lax.fori_loop(0, nb, block, 0)

    return k(x2, pos2)


def kernel(x, pos_table):
    B, S, D = x.shape
    rpw = B * S // _NW
    wpb = _NW // B
    rc = 32
    nb = rpw // rc
    x2 = x.reshape(B * S, D)
    pos2 = pos_table.reshape(S, D)
    out = _sc_add(x2, pos2, rpw=rpw, wpb=wpb, rc=rc, nb=nb, d=D)
    return out.reshape(B, S, D)
